# baseline (speedup 1.0000x reference)
import functools

import jax
import jax.numpy as jnp
from jax import lax
from jax.experimental import pallas as pl
from jax.experimental.pallas import tpu as pltpu
from jax.experimental.pallas import tpu_sc as plsc

B = 4096
S = 20
E = 128
HID = 1024
NCLS = 2

NC = 2
NS = 16
NW = NC * NS
CHUNK = 4
IDX_PER_CHUNK = CHUNK * S
N_CHUNKS_TOT = B // CHUNK
L = 16
NBUF = 6

SPLIT1 = 3072
SPLIT2 = B - SPLIT1

BB = 1024


def _gather_mean_body(b_per_w, chunk_off, idx_hbm, table_hbm, out_hbm,
                      idx_v, rows_v, out_v, sem):
    n_chunks = b_per_w // CHUNK
    wid = lax.axis_index("s") * NC + lax.axis_index("c")
    pltpu.sync_copy(
        idx_hbm.at[pl.ds(chunk_off + wid * n_chunks, n_chunks)], idx_v)
    for p in range(NBUF - 1):
        pltpu.async_copy(table_hbm.at[idx_v.at[p]], rows_v.at[p], sem)

    inv_s = jnp.float32(1.0 / S)

    def chunk_body(c, _):
        buf = lax.rem(c, NBUF)
        nxt = lax.rem(c + NBUF - 1, NBUF)

        @pl.when(c + NBUF - 1 < n_chunks)
        def _prefetch():
            pltpu.async_copy(table_hbm.at[idx_v.at[c + NBUF - 1]],
                             rows_v.at[nxt], sem)

        pltpu.make_async_copy(
            table_hbm.at[idx_v.at[c]], rows_v.at[buf], sem
        ).wait()

        for r in range(CHUNK):
            base = r * S
            for g in range(E // L):
                sl = pl.ds(g * L, L)
                a0 = rows_v[buf, base + 0, sl]
                a1 = rows_v[buf, base + 1, sl]
                a2 = rows_v[buf, base + 2, sl]
                a3 = rows_v[buf, base + 3, sl]
                for j in range(4, S, 4):
                    a0 = a0 + rows_v[buf, base + j + 0, sl]
                    a1 = a1 + rows_v[buf, base + j + 1, sl]
                    a2 = a2 + rows_v[buf, base + j + 2, sl]
                    a3 = a3 + rows_v[buf, base + j + 3, sl]
                out_v[c * CHUNK + r, sl] = ((a0 + a1) + (a2 + a3)) * inv_s
        return 0

    lax.fori_loop(0, n_chunks, chunk_body, 0)
    pltpu.sync_copy(out_v, out_hbm.at[pl.ds(wid * b_per_w, b_per_w)])


@functools.cache
def _gather_mean(bslice, chunk_off):
    b_per_w = bslice // NW
    n_chunks = b_per_w // CHUNK
    mesh = plsc.VectorSubcoreMesh(core_axis_name="c", subcore_axis_name="s")
    return pl.kernel(
        functools.partial(_gather_mean_body, b_per_w, chunk_off),
        mesh=mesh,
        out_type=jax.ShapeDtypeStruct((bslice, E), jnp.float32),
        scratch_types=[
            pltpu.VMEM((n_chunks, IDX_PER_CHUNK), jnp.int32),
            pltpu.VMEM((NBUF, IDX_PER_CHUNK, E), jnp.float32),
            pltpu.VMEM((b_per_w, E), jnp.float32),
            pltpu.SemaphoreType.DMA,
        ],
    )


def _mlp_math(x, w1, b1, w2, b2, w3, b3):
    dn = (((1,), (1,)), ((), ()))
    h = lax.dot_general(x, w1, dn, preferred_element_type=jnp.float32)
    h = jnp.maximum(h + b1, 0.0)
    h = lax.dot_general(h, w2, dn, preferred_element_type=jnp.float32)
    h = jnp.maximum(h + b2, 0.0)
    logits = lax.dot_general(h, w3, dn, preferred_element_type=jnp.float32)
    logits = logits + b3
    m = jnp.max(logits, axis=-1, keepdims=True)
    sh = logits - m
    lse = jnp.log(jnp.sum(jnp.exp(sh), axis=-1, keepdims=True))
    return sh - lse


def _mlp_body(x_ref, w1_ref, b1_ref, w2_ref, b2_ref, w3_ref, b3_ref, o_ref):
    o_ref[...] = _mlp_math(x_ref[...], w1_ref[...], b1_ref[...], w2_ref[...],
                           b2_ref[...], w3_ref[...], b3_ref[...])


def _mlp(avg, W1, b1, W2, b2, W3, b3):
    bsz = avg.shape[0]
    grid = (bsz // BB,)
    return pl.pallas_call(
        _mlp_body,
        grid=grid,
        in_specs=[
            pl.BlockSpec((BB, E), lambda i: (i, 0)),
            pl.BlockSpec((HID, E), lambda i: (0, 0)),
            pl.BlockSpec((1, HID), lambda i: (0, 0)),
            pl.BlockSpec((HID, HID), lambda i: (0, 0)),
            pl.BlockSpec((1, HID), lambda i: (0, 0)),
            pl.BlockSpec((NCLS, HID), lambda i: (0, 0)),
            pl.BlockSpec((1, NCLS), lambda i: (0, 0)),
        ],
        out_specs=pl.BlockSpec((BB, NCLS), lambda i: (i, 0)),
        out_shape=jax.ShapeDtypeStruct((bsz, NCLS), jnp.float32),
    )(avg, W1, b1, W2, b2, W3, b3)


N1 = SPLIT1 // BB
N2 = SPLIT2 // BB


def _mlp_tail_body(out1_ref, x_ref, w1_ref, b1_ref, w2_ref, b2_ref, w3_ref,
                   b3_ref, o_ref):
    i = pl.program_id(0)

    @pl.when(i < N1)
    def _copy():
        o_ref[...] = out1_ref[...]

    @pl.when(i >= N1)
    def _compute():
        o_ref[...] = _mlp_math(x_ref[...], w1_ref[...], b1_ref[...],
                               w2_ref[...], b2_ref[...], w3_ref[...],
                               b3_ref[...])


def _mlp_tail(out1, avg2, W1, b1, W2, b2, W3, b3):
    grid = (N1 + N2,)
    return pl.pallas_call(
        _mlp_tail_body,
        grid=grid,
        in_specs=[
            pl.BlockSpec((BB, NCLS), lambda i: (jnp.minimum(i, N1 - 1), 0)),
            pl.BlockSpec((BB, E), lambda i: (jnp.maximum(i - N1, 0), 0)),
            pl.BlockSpec((HID, E), lambda i: (0, 0)),
            pl.BlockSpec((1, HID), lambda i: (0, 0)),
            pl.BlockSpec((HID, HID), lambda i: (0, 0)),
            pl.BlockSpec((1, HID), lambda i: (0, 0)),
            pl.BlockSpec((NCLS, HID), lambda i: (0, 0)),
            pl.BlockSpec((1, NCLS), lambda i: (0, 0)),
        ],
        out_specs=pl.BlockSpec((BB, NCLS), lambda i: (i, 0)),
        out_shape=jax.ShapeDtypeStruct((B, NCLS), jnp.float32),
    )(out1, avg2, W1, b1, W2, b2, W3, b3)


def kernel(word_indices, emb_table, W1, b1, W2, b2, W3, b3):
    idx = word_indices.reshape(N_CHUNKS_TOT, IDX_PER_CHUNK).astype(jnp.int32)
    b1r = b1.reshape(1, HID)
    b2r = b2.reshape(1, HID)
    b3r = b3.reshape(1, NCLS)
    avg1 = _gather_mean(SPLIT1, 0)(idx, emb_table)
    avg2 = _gather_mean(SPLIT2, SPLIT1 // CHUNK)(idx, emb_table)
    out1 = _mlp(avg1, W1, b1r, W2, b2r, W3, b3r)
    return _mlp_tail(out1, avg2, W1, b1r, W2, b2r, W3, b3r)
